# baseline (speedup 1.0000x reference)
import functools

import jax
import jax.numpy as jnp
from jax import lax
from jax.experimental import pallas as pl
from jax.experimental.pallas import tpu as pltpu
from jax.experimental.pallas import tpu_sc as plsc

_B, _T, _N, _D = 8, 4, 1025, 768
_BT = _B * _T
_SLAB = _N * _D
_SROWS = 32
_STRIPE = _SROWS * _D
_TAIL_OFF = (_N - 1) * _D
_NVEC = _STRIPE // 16
_NVEC_TAIL = _D // 16


def _sc_body(x_hbm, ar_hbm, local_hbm, glob_hbm, gate_hbm, out_hbm,
             xb0, xb1, xb2, gbuf, lbuf, xtail, ltail, gtail, arv_b, gv_b,
             xsem, osem, gsem, tsem):
    xbufs = (xb0, xb1, xb2)
    wid = lax.axis_index("s") * 2 + lax.axis_index("c")
    stripe_off = wid * _STRIPE

    pltpu.sync_copy(ar_hbm, arv_b)
    pltpu.sync_copy(gate_hbm, gv_b)
    pltpu.sync_copy(local_hbm.at[pl.ds(stripe_off, _STRIPE)], lbuf)
    pltpu.sync_copy(local_hbm.at[pl.ds(_TAIL_OFF, _D)], ltail)

    gv = gv_b[...]
    arv = arv_b[...]
    tgv = 1.0 - 2.0 / (jnp.exp(2.0 * gv) + 1.0)
    lsv = 1.0 - tgv
    gate_nz = gv[0] != 0.0

    def _zero(i, _):
        gbuf[pl.ds(i * 16, 16)] = jnp.zeros((16,), jnp.float32)
        return 0
    lax.fori_loop(0, _NVEC, _zero, 0, unroll=8)

    def _zero_t(i, _):
        gtail[pl.ds(i * 16, 16)] = jnp.zeros((16,), jnp.float32)
        return 0
    lax.fori_loop(0, _NVEC_TAIL, _zero_t, 0)

    def slab_meta(b, t):
        h = arv[2 * b]
        w = arv[2 * b + 1]
        mask = t < h * w
        use = jnp.logical_and(mask, gate_nz)
        safe_w = jnp.maximum(w, 1)
        gidx = (t // safe_w) * 4 + t % safe_w
        coefv = tgv * jnp.full((16,), use.astype(jnp.float32))
        return use, coefv, gidx

    def x_in(s, slot):
        return pltpu.make_async_copy(
            x_hbm.at[pl.ds(s * _SLAB + stripe_off, _STRIPE)], xbufs[slot],
            xsem.at[slot])

    def x_out(s, slot):
        return pltpu.make_async_copy(
            xbufs[slot], out_hbm.at[pl.ds(s * _SLAB + stripe_off, _STRIPE)],
            osem.at[slot])

    x_in(0, 0).start()
    for s in range(_BT):
        slot = s % 3
        if s + 1 < _BT:
            nslot = (s + 1) % 3
            if s + 1 >= 3:
                x_out(s - 2, nslot).wait()
            x_in(s + 1, nslot).start()
        x_in(s, slot).wait()

        use, coefv, gidx = slab_meta(s // 4, s % 4)

        @pl.when(use)
        def _(gidx=gidx):
            pltpu.make_async_copy(
                glob_hbm.at[pl.ds(gidx * _SLAB + stripe_off, _STRIPE)], gbuf,
                gsem).start()
            pltpu.make_async_copy(
                glob_hbm.at[pl.ds(gidx * _SLAB + stripe_off, _STRIPE)], gbuf,
                gsem).wait()

        xb = xbufs[slot]

        def _step(i, _, xb=xb, coefv=coefv):
            sl = pl.ds(i * 16, 16)
            xb[sl] = xb[sl] + lbuf[sl] * lsv + gbuf[sl] * coefv
            return 0
        lax.fori_loop(0, _NVEC, _step, 0, unroll=4)
        x_out(s, slot).start()

        @pl.when(wid == s)
        def _(s=s, use=use, coefv=coefv, gidx=gidx):
            tail_base = s * _SLAB + _TAIL_OFF
            pltpu.sync_copy(x_hbm.at[pl.ds(tail_base, _D)], xtail)

            @pl.when(use)
            def _():
                pltpu.make_async_copy(
                    glob_hbm.at[pl.ds(gidx * _SLAB + _TAIL_OFF, _D)], gtail,
                    tsem).start()
                pltpu.make_async_copy(
                    glob_hbm.at[pl.ds(gidx * _SLAB + _TAIL_OFF, _D)], gtail,
                    tsem).wait()

            def _step_t(i, _):
                sl = pl.ds(i * 16, 16)
                xtail[sl] = xtail[sl] + ltail[sl] * lsv + gtail[sl] * coefv
                return 0
            lax.fori_loop(0, _NVEC_TAIL, _step_t, 0, unroll=8)
            pltpu.sync_copy(xtail, out_hbm.at[pl.ds(tail_base, _D)])

    for s in range(_BT - 3, _BT):
        x_out(s, s % 3).wait()


def _sc_call(x_flat, ar_flat, local_flat, glob_flat, gate16):
    mesh = plsc.VectorSubcoreMesh(core_axis_name="c", subcore_axis_name="s")
    f = pl.kernel(
        _sc_body,
        mesh=mesh,
        out_type=jax.ShapeDtypeStruct((_BT * _SLAB,), jnp.float32),
        scratch_types=[
            pltpu.VMEM((_STRIPE,), jnp.float32),
            pltpu.VMEM((_STRIPE,), jnp.float32),
            pltpu.VMEM((_STRIPE,), jnp.float32),
            pltpu.VMEM((_STRIPE,), jnp.float32),
            pltpu.VMEM((_STRIPE,), jnp.float32),
            pltpu.VMEM((_D,), jnp.float32),
            pltpu.VMEM((_D,), jnp.float32),
            pltpu.VMEM((_D,), jnp.float32),
            pltpu.VMEM((16,), jnp.int32),
            pltpu.VMEM((16,), jnp.float32),
            pltpu.SemaphoreType.DMA((3,)),
            pltpu.SemaphoreType.DMA((3,)),
            pltpu.SemaphoreType.DMA,
            pltpu.SemaphoreType.DMA,
        ],
    )
    return f(x_flat, ar_flat, local_flat, glob_flat, gate16)


def kernel(x, aspect_ratio, local_token_positional_embedding,
           global_token_positional_embedding, gate):
    B, T, N, D = x.shape
    x_flat = x.reshape(-1)
    ar_flat = jnp.broadcast_to(
        aspect_ratio.astype(jnp.int32).reshape(-1), (16,))
    local_flat = local_token_positional_embedding.reshape(-1)
    glob_flat = global_token_positional_embedding.reshape(-1)
    gate16 = jnp.broadcast_to(gate.astype(jnp.float32), (16,))
    out = _sc_call(x_flat, ar_flat, local_flat, glob_flat, gate16)
    return out.reshape(B, T, N, D)


# speedup vs baseline: 1.1190x; 1.1190x over previous
import jax
import jax.numpy as jnp
from jax import lax
from jax.experimental import pallas as pl
from jax.experimental.pallas import tpu as pltpu
from jax.experimental.pallas import tpu_sc as plsc

_B, _T, _N, _D = 8, 4, 1025, 768
_BT = _B * _T
_SROWS = 32
_NCOL = _D // 16


def _sc_body(x_hbm, ar_hbm, local_hbm, glob_hbm, gate_hbm, out_hbm,
             xb0, xb1, xb2, gbuf, lbuf, xtail, ltail, gtail, arv_b, gv_b,
             xsem, osem, gsem, tsem):
    xbufs = (xb0, xb1, xb2)
    wid = lax.axis_index("s") * 2 + lax.axis_index("c")
    row0 = pl.multiple_of(wid * _SROWS, _SROWS)

    pltpu.sync_copy(ar_hbm, arv_b)
    pltpu.sync_copy(gate_hbm, gv_b)
    pltpu.sync_copy(local_hbm.at[pl.ds(row0, _SROWS), :], lbuf)
    pltpu.sync_copy(local_hbm.at[pl.ds(_N - 1, 1), :], ltail)

    gv = gv_b[...]
    arv = arv_b[...]
    tgv = 1.0 - 2.0 / (jnp.exp(2.0 * gv) + 1.0)
    lsv = 1.0 - tgv
    gate_nz = gv[0] != 0.0

    def _zero_r(r, _):
        def _zero_c(c, _):
            gbuf[r, pl.ds(c * 16, 16)] = jnp.zeros((16,), jnp.float32)
            return 0
        lax.fori_loop(0, _NCOL, _zero_c, 0, unroll=4)
        return 0
    lax.fori_loop(0, _SROWS, _zero_r, 0)

    def _zero_t(c, _):
        gtail[0, pl.ds(c * 16, 16)] = jnp.zeros((16,), jnp.float32)
        return 0
    lax.fori_loop(0, _NCOL, _zero_t, 0, unroll=4)

    def slab_meta(b, t):
        h = arv[2 * b]
        w = arv[2 * b + 1]
        mask = t < h * w
        use = jnp.logical_and(mask, gate_nz)
        safe_w = jnp.maximum(w, 1)
        gidx = (t // safe_w) * 4 + t % safe_w
        coefv = tgv * jnp.full((16,), use.astype(jnp.float32))
        return use, coefv, gidx

    def x_in(s, slot):
        return pltpu.make_async_copy(
            x_hbm.at[s, pl.ds(row0, _SROWS), :], xbufs[slot], xsem.at[slot])

    def x_out(s, slot):
        return pltpu.make_async_copy(
            xbufs[slot], out_hbm.at[s, pl.ds(row0, _SROWS), :], osem.at[slot])

    x_in(0, 0).start()
    for s in range(_BT):
        slot = s % 3
        if s + 1 < _BT:
            nslot = (s + 1) % 3
            if s + 1 >= 3:
                x_out(s - 2, nslot).wait()
            x_in(s + 1, nslot).start()
        x_in(s, slot).wait()

        use, coefv, gidx = slab_meta(s // 4, s % 4)

        @pl.when(use)
        def _(gidx=gidx):
            pltpu.make_async_copy(
                glob_hbm.at[gidx, pl.ds(row0, _SROWS), :], gbuf, gsem).start()
            pltpu.make_async_copy(
                glob_hbm.at[gidx, pl.ds(row0, _SROWS), :], gbuf, gsem).wait()

        xb = xbufs[slot]

        def _row(r, _, xb=xb, coefv=coefv):
            def _col(c, _, xb=xb, coefv=coefv):
                sl = pl.ds(c * 16, 16)
                xb[r, sl] = xb[r, sl] + lbuf[r, sl] * lsv + gbuf[r, sl] * coefv
                return 0
            lax.fori_loop(0, _NCOL, _col, 0, unroll=4)
            return 0
        lax.fori_loop(0, _SROWS, _row, 0)
        x_out(s, slot).start()

        @pl.when(wid == s)
        def _(s=s, use=use, coefv=coefv, gidx=gidx):
            pltpu.sync_copy(x_hbm.at[s, pl.ds(_N - 1, 1), :], xtail)

            @pl.when(use)
            def _():
                pltpu.make_async_copy(
                    glob_hbm.at[gidx, pl.ds(_N - 1, 1), :], gtail,
                    tsem).start()
                pltpu.make_async_copy(
                    glob_hbm.at[gidx, pl.ds(_N - 1, 1), :], gtail,
                    tsem).wait()

            def _col_t(c, _):
                sl = pl.ds(c * 16, 16)
                xtail[0, sl] = (xtail[0, sl] + ltail[0, sl] * lsv
                                + gtail[0, sl] * coefv)
                return 0
            lax.fori_loop(0, _NCOL, _col_t, 0, unroll=4)
            pltpu.sync_copy(xtail, out_hbm.at[s, pl.ds(_N - 1, 1), :])

    for s in range(_BT - 3, _BT):
        x_out(s, s % 3).wait()


def _sc_call(x3, ar16, local2, glob3, gate16):
    mesh = plsc.VectorSubcoreMesh(core_axis_name="c", subcore_axis_name="s")
    f = pl.kernel(
        _sc_body,
        mesh=mesh,
        out_type=jax.ShapeDtypeStruct((_BT, _N, _D), jnp.float32),
        scratch_types=[
            pltpu.VMEM((_SROWS, _D), jnp.float32),
            pltpu.VMEM((_SROWS, _D), jnp.float32),
            pltpu.VMEM((_SROWS, _D), jnp.float32),
            pltpu.VMEM((_SROWS, _D), jnp.float32),
            pltpu.VMEM((_SROWS, _D), jnp.float32),
            pltpu.VMEM((1, _D), jnp.float32),
            pltpu.VMEM((1, _D), jnp.float32),
            pltpu.VMEM((1, _D), jnp.float32),
            pltpu.VMEM((16,), jnp.int32),
            pltpu.VMEM((16,), jnp.float32),
            pltpu.SemaphoreType.DMA((3,)),
            pltpu.SemaphoreType.DMA((3,)),
            pltpu.SemaphoreType.DMA,
            pltpu.SemaphoreType.DMA,
        ],
    )
    return f(x3, ar16, local2, glob3, gate16)


def kernel(x, aspect_ratio, local_token_positional_embedding,
           global_token_positional_embedding, gate):
    B, T, N, D = x.shape
    x3 = x.reshape(B * T, N, D)
    glob3 = global_token_positional_embedding.reshape(-1, N, D)
    ar16 = jnp.broadcast_to(aspect_ratio.astype(jnp.int32).reshape(-1), (16,))
    gate16 = jnp.broadcast_to(gate.astype(jnp.float32), (16,))
    out = _sc_call(x3, ar16, local_token_positional_embedding, glob3, gate16)
    return out.reshape(B, T, N, D)


# speedup vs baseline: 8.8721x; 7.9282x over previous
import jax
import jax.numpy as jnp
from jax import lax
from jax.experimental import pallas as pl
from jax.experimental.pallas import tpu as pltpu
from jax.experimental.pallas import tpu_sc as plsc

_B, _T, _N, _D = 8, 4, 1025, 768
_NV = _D // 16
_QTOK = (_N - 1) // 4
_CH = 8
_NCHUNK = _QTOK // _CH


def _sc_body(x_hbm, ar_hbm, local_hbm, glob_hbm, gate_hbm, ltail_hbm,
             gtail_hbm, out_hbm,
             xb0, xb1, xb2, gbuf, lbuf, xt, lt, gt, arv_b, gv_b,
             xsem, osem, lsem, gsem):
    xbufs = (xb0, xb1, xb2)
    wid = lax.axis_index("s") * 2 + lax.axis_index("c")
    b = wid // 4
    q = wid % 4
    tok_base = q * _QTOK

    pltpu.sync_copy(ar_hbm.at[pl.ds(pl.multiple_of(16 * b, 16), 16)], arv_b)
    pltpu.sync_copy(gate_hbm, gv_b)

    gv = gv_b[...]
    tgv = 1.0 - 2.0 / (jnp.exp(2.0 * gv) + 1.0)
    lsv = 1.0 - tgv
    gate_nz = gv[0] != 0.0

    arv = arv_b[...]
    h = arv[0]
    w = arv[1]
    safe_w = jnp.maximum(w, 1)
    metas = []
    for t in range(_T):
        mask_t = t < h * w
        use_t = jnp.logical_and(mask_t, gate_nz)
        gh_t = t // safe_w
        gw_t = t % safe_w
        coefv_t = tgv * jnp.full((16,), use_t.astype(jnp.float32))
        metas.append((use_t, coefv_t, gh_t, gw_t))

    def x_in(c, slot):
        return pltpu.make_async_copy(
            x_hbm.at[b, pl.ds(tok_base + c * _CH, _CH), :, :], xbufs[slot],
            xsem.at[slot])

    def x_out(c, slot):
        return pltpu.make_async_copy(
            xbufs[slot], out_hbm.at[b, pl.ds(tok_base + c * _CH, _CH), :, :],
            osem.at[slot])

    def do_chunk(c, slot):
        pltpu.make_async_copy(
            local_hbm.at[pl.ds(tok_base + c * _CH, _CH), :], lbuf,
            lsem).start()
        x_in(c, slot).wait()
        pltpu.make_async_copy(
            local_hbm.at[pl.ds(tok_base + c * _CH, _CH), :], lbuf,
            lsem).wait()
        xb = xbufs[slot]

        def _nn_body(nn, _, xb=xb):
            def _j_body(j, _, xb=xb):
                sl = pl.ds(j * 16, 16)
                lvec = lbuf[nn, sl] * lsv
                for t in range(_T):
                    xb[nn, t, sl] = xb[nn, t, sl] + lvec
                return 0
            lax.fori_loop(0, _NV, _j_body, 0, unroll=4)
            return 0
        lax.fori_loop(0, _CH, _nn_body, 0)

        for gh in range(2):
            need = jnp.logical_and(metas[0][0], metas[0][2] == gh)
            for t in range(1, _T):
                need = jnp.logical_or(
                    need, jnp.logical_and(metas[t][0], metas[t][2] == gh))

            @pl.when(need)
            def _(gh=gh, c=c, xb=xb):
                pltpu.make_async_copy(
                    glob_hbm.at[gh, pl.ds(tok_base + c * _CH, _CH), :, :],
                    gbuf, gsem).start()
                pltpu.make_async_copy(
                    glob_hbm.at[gh, pl.ds(tok_base + c * _CH, _CH), :, :],
                    gbuf, gsem).wait()
                for t in range(_T):
                    use_t, coefv_t, gh_t, gw_t = metas[t]

                    @pl.when(jnp.logical_and(use_t, gh_t == gh))
                    def _(t=t, coefv_t=coefv_t, gw_t=gw_t, xb=xb):
                        def _nn_g(nn, _, xb=xb):
                            def _j_g(j, _, xb=xb):
                                sl = pl.ds(j * 16, 16)
                                xb[nn, t, sl] = (xb[nn, t, sl]
                                                 + gbuf[nn, gw_t, sl]
                                                 * coefv_t)
                                return 0
                            lax.fori_loop(0, _NV, _j_g, 0, unroll=4)
                            return 0
                        lax.fori_loop(0, _CH, _nn_g, 0)

        x_out(c, slot).start()

    x_in(0, 0).start()

    def group(g, _):
        for k in range(3):
            c = 3 * g + k
            slot = k
            nslot = (k + 1) % 3

            @pl.when(c + 1 < _NCHUNK)
            def _(c=c, nslot=nslot):
                @pl.when(c + 1 >= 3)
                def _():
                    x_out(c - 2, nslot).wait()
                x_in(c + 1, nslot).start()

            do_chunk(c, slot)
        return 0

    lax.fori_loop(0, _NCHUNK // 3, group, 0)
    for c in (_NCHUNK - 2, _NCHUNK - 1):
        slot = c % 3
        if c + 1 < _NCHUNK:
            nslot = (c + 1) % 3
            x_out(c - 2, nslot).wait()
            x_in(c + 1, nslot).start()
        do_chunk(c, slot)
    for c in range(_NCHUNK - 3, _NCHUNK):
        x_out(c, c % 3).wait()

    @pl.when(q == 3)
    def _():
        pltpu.sync_copy(x_hbm.at[b, pl.ds(_N - 1, 1), :, :], xt)
        pltpu.sync_copy(ltail_hbm, lt)

        def _j_t(j, _):
            sl = pl.ds(j * 16, 16)
            lvec = lt[0, sl] * lsv
            for t in range(_T):
                xt[0, t, sl] = xt[0, t, sl] + lvec
            return 0
        lax.fori_loop(0, _NV, _j_t, 0, unroll=4)

        for t in range(_T):
            use_t, coefv_t, gh_t, gw_t = metas[t]

            @pl.when(use_t)
            def _(t=t, coefv_t=coefv_t, gh_t=gh_t, gw_t=gw_t):
                gi = gh_t * 4 + gw_t
                pltpu.sync_copy(
                    gtail_hbm.at[pl.ds(0, 8), pl.ds(gi * _D, _D)], gt)

                def _j_gt(j, _):
                    sl = pl.ds(j * 16, 16)
                    xt[0, t, sl] = xt[0, t, sl] + gt[0, sl] * coefv_t
                    return 0
                lax.fori_loop(0, _NV, _j_gt, 0, unroll=4)

        pltpu.sync_copy(xt, out_hbm.at[b, pl.ds(_N - 1, 1), :, :])


def _sc_call(xv, arpk, local2, globv, gate16, ltail, gtail):
    mesh = plsc.VectorSubcoreMesh(core_axis_name="c", subcore_axis_name="s")
    f = pl.kernel(
        _sc_body,
        mesh=mesh,
        out_type=jax.ShapeDtypeStruct((_B, _N, _T, _D), jnp.float32),
        scratch_types=[
            pltpu.VMEM((_CH, _T, _D), jnp.float32),
            pltpu.VMEM((_CH, _T, _D), jnp.float32),
            pltpu.VMEM((_CH, _T, _D), jnp.float32),
            pltpu.VMEM((_CH, _T, _D), jnp.float32),
            pltpu.VMEM((_CH, _D), jnp.float32),
            pltpu.VMEM((1, _T, _D), jnp.float32),
            pltpu.VMEM((8, _D), jnp.float32),
            pltpu.VMEM((8, _D), jnp.float32),
            pltpu.VMEM((16,), jnp.int32),
            pltpu.VMEM((16,), jnp.float32),
            pltpu.SemaphoreType.DMA((3,)),
            pltpu.SemaphoreType.DMA((3,)),
            pltpu.SemaphoreType.DMA,
            pltpu.SemaphoreType.DMA,
        ],
    )
    return f(xv, arpk, local2, globv, gate16, ltail, gtail)


def kernel(x, aspect_ratio, local_token_positional_embedding,
           global_token_positional_embedding, gate):
    B, T, N, D = x.shape
    local = local_token_positional_embedding
    glob = global_token_positional_embedding
    xv = jnp.transpose(x, (0, 2, 1, 3))
    globv = jnp.transpose(glob, (0, 2, 1, 3))
    arpk = jnp.pad(aspect_ratio.astype(jnp.int32),
                   ((0, 0), (0, 14))).reshape(-1)
    gate16 = jnp.broadcast_to(gate.astype(jnp.float32), (16,))
    ltail = jnp.broadcast_to(local[N - 1].reshape(1, D), (8, D))
    gtail = jnp.broadcast_to(
        glob[:, :, N - 1, :].reshape(1, 16 * D), (8, 16 * D))
    outv = _sc_call(xv, arpk, local, globv, gate16, ltail, gtail)
    return jnp.transpose(outv, (0, 2, 1, 3))
